# baseline (speedup 1.0000x reference)
import functools

import jax
import jax.numpy as jnp
from jax import lax
from jax.experimental import pallas as pl
from jax.experimental.pallas import tpu as pltpu
from jax.experimental.pallas import tpu_sc as plsc

EMB_D = 64
IDX_ROW = 128
KSUB = 4
CHUNK = KSUB * IDX_ROW


K1_COLS = 2048


def _expand_table(weight_t, tail_tab):
    n = weight_t.shape[1]
    n_full = (n // K1_COLS) * K1_COLS
    steps = n_full // K1_COLS + 1

    def body(in_ref, tail_ref, out_ref):
        r = pl.program_id(0)

        @pl.when(r < steps - 1)
        def _():
            t = in_ref[...].T
            out_ref[...] = jnp.concatenate([t, t], axis=1)

        @pl.when(r == steps - 1)
        def _():
            out_ref[...] = tail_ref[...]

    return pl.pallas_call(
        body,
        grid=(steps,),
        in_specs=[
            pl.BlockSpec(
                (EMB_D, K1_COLS), lambda r: (0, jnp.minimum(r, steps - 2))
            ),
            pl.BlockSpec((K1_COLS, 2 * EMB_D), lambda r: (0, 0)),
        ],
        out_specs=pl.BlockSpec((K1_COLS, 2 * EMB_D), lambda r: (r, 0)),
        out_shape=jax.ShapeDtypeStruct((steps * K1_COLS, 2 * EMB_D), jnp.float32),
    )(weight_t, tail_tab)


def _gather_sc(weight, idx_rows, b_total):
    info = plsc.get_sparse_core_info()
    nc, ns = info.num_cores, info.num_subcores
    nw = nc * ns
    rows_per_w = b_total // nw
    g_steps = rows_per_w // CHUNK
    irows_per_w = rows_per_w // IDX_ROW

    mesh = plsc.VectorSubcoreMesh(core_axis_name="c", subcore_axis_name="s")

    @functools.partial(
        pl.kernel,
        mesh=mesh,
        compiler_params=pltpu.CompilerParams(use_tc_tiling_on_sc=False),
        out_type=jax.ShapeDtypeStruct((b_total, EMB_D), jnp.float32),
        scratch_types=[
            pltpu.VMEM((irows_per_w, IDX_ROW), jnp.int32),
            pltpu.VMEM((CHUNK, EMB_D), jnp.float32),
            pltpu.VMEM((CHUNK, EMB_D), jnp.float32),
            pltpu.SemaphoreType.DMA,
            pltpu.SemaphoreType.DMA,
            pltpu.SemaphoreType.DMA,
            pltpu.SemaphoreType.DMA,
        ],
    )
    def k(table_hbm, idx_hbm, out_hbm, idx_v, rows0, rows1, gs0, gs1, ss0, ss1):
        wid = lax.axis_index("s") * nc + lax.axis_index("c")
        irow_base = wid * irows_per_w
        out_base = wid * rows_per_w

        pltpu.sync_copy(idx_hbm.at[pl.ds(irow_base, irows_per_w)], idx_v)

        def fire_gather(chunk, buf, sem):
            for j in range(KSUB):
                pltpu.async_copy(
                    table_hbm.at[idx_v.at[chunk * KSUB + j]],
                    buf.at[pl.ds(j * IDX_ROW, IDX_ROW)],
                    sem,
                )

        def wait_gather(chunk, buf, sem):
            for j in range(KSUB):
                pltpu.make_async_copy(
                    table_hbm.at[idx_v.at[chunk * KSUB + j]],
                    buf.at[pl.ds(j * IDX_ROW, IDX_ROW)],
                    sem,
                ).wait()

        def fire_store(chunk, buf, sem):
            pltpu.async_copy(
                buf, out_hbm.at[pl.ds(out_base + chunk * CHUNK, CHUNK)], sem
            )

        def wait_store(chunk, buf, sem):
            pltpu.make_async_copy(
                buf, out_hbm.at[pl.ds(out_base + chunk * CHUNK, CHUNK)], sem
            ).wait()

        fire_gather(0, rows0, gs0)

        t_steps = g_steps // 2

        def body(t, carry):
            a = 2 * t
            b = a + 1

            @pl.when(t > 0)
            def _():
                wait_store(a - 1, rows1, ss1)

            fire_gather(b, rows1, gs1)
            wait_gather(a, rows0, gs0)
            fire_store(a, rows0, ss0)

            @pl.when(t + 1 < t_steps)
            def _():
                wait_store(a, rows0, ss0)
                fire_gather(a + 2, rows0, gs0)

            wait_gather(b, rows1, gs1)
            fire_store(b, rows1, ss1)
            return carry

        lax.fori_loop(0, t_steps, body, 0)
        wait_store(g_steps - 2, rows0, ss0)
        wait_store(g_steps - 1, rows1, ss1)

    return k(weight, idx_rows)


def kernel(token_ids, weight):
    b, s = token_ids.shape
    b_total = b * s
    n = weight.shape[0]
    n_full = (n // K1_COLS) * K1_COLS
    tail_tab = jnp.pad(
        weight[n_full:], ((0, K1_COLS - (n - n_full)), (0, EMB_D))
    )
    table = _expand_table(weight.T, tail_tab)
    wpad = table.reshape(2 * table.shape[0], EMB_D)
    idx_rows = (token_ids.astype(jnp.int32) * 2).reshape(b_total // IDX_ROW, IDX_ROW)
    out = _gather_sc(wpad, idx_rows, b_total)
    return out.reshape(b, s, EMB_D)


# speedup vs baseline: 1.1897x; 1.1897x over previous
import functools

import jax
import jax.numpy as jnp
from jax import lax
from jax.experimental import pallas as pl
from jax.experimental.pallas import tpu as pltpu
from jax.experimental.pallas import tpu_sc as plsc

EMB_D = 64
IDX_ROW = 128
KSUB = 4
CHUNK = KSUB * IDX_ROW


K1_COLS = 8192


def _expand_table(weight_t, tail_tab):
    n = weight_t.shape[1]
    n_full = (n // K1_COLS) * K1_COLS
    steps = n_full // K1_COLS + 1

    def body(in_ref, tail_ref, out_ref):
        r = pl.program_id(0)

        @pl.when(r < steps - 1)
        def _():
            t = in_ref[...].T
            out_ref[...] = jnp.concatenate([t, t], axis=1)

        @pl.when(r == steps - 1)
        def _():
            out_ref[...] = tail_ref[...]

    return pl.pallas_call(
        body,
        grid=(steps,),
        in_specs=[
            pl.BlockSpec(
                (EMB_D, K1_COLS), lambda r: (0, jnp.minimum(r, steps - 2))
            ),
            pl.BlockSpec((K1_COLS, 2 * EMB_D), lambda r: (0, 0)),
        ],
        out_specs=pl.BlockSpec((K1_COLS, 2 * EMB_D), lambda r: (r, 0)),
        out_shape=jax.ShapeDtypeStruct((steps * K1_COLS, 2 * EMB_D), jnp.float32),
    )(weight_t, tail_tab)


def _gather_sc(weight, idx_rows, b_total):
    info = plsc.get_sparse_core_info()
    nc, ns = info.num_cores, info.num_subcores
    nw = nc * ns
    rows_per_w = b_total // nw
    g_steps = rows_per_w // CHUNK
    irows_per_w = rows_per_w // IDX_ROW

    mesh = plsc.VectorSubcoreMesh(core_axis_name="c", subcore_axis_name="s")

    @functools.partial(
        pl.kernel,
        mesh=mesh,
        compiler_params=pltpu.CompilerParams(use_tc_tiling_on_sc=False),
        out_type=jax.ShapeDtypeStruct((b_total, EMB_D), jnp.float32),
        scratch_types=[
            pltpu.VMEM((irows_per_w, IDX_ROW), jnp.int32),
            pltpu.VMEM((CHUNK, EMB_D), jnp.float32),
            pltpu.VMEM((CHUNK, EMB_D), jnp.float32),
            pltpu.SemaphoreType.DMA,
            pltpu.SemaphoreType.DMA,
            pltpu.SemaphoreType.DMA,
            pltpu.SemaphoreType.DMA,
        ],
    )
    def k(table_hbm, idx_hbm, out_hbm, idx_v, rows0, rows1, gs0, gs1, ss0, ss1):
        wid = lax.axis_index("s") * nc + lax.axis_index("c")
        irow_base = wid * irows_per_w
        out_base = wid * rows_per_w

        pltpu.sync_copy(idx_hbm.at[pl.ds(irow_base, irows_per_w)], idx_v)

        def fire_gather(chunk, buf, sem):
            for j in range(KSUB):
                pltpu.async_copy(
                    table_hbm.at[idx_v.at[chunk * KSUB + j]],
                    buf.at[pl.ds(j * IDX_ROW, IDX_ROW)],
                    sem,
                )

        def wait_gather(chunk, buf, sem):
            for j in range(KSUB):
                pltpu.make_async_copy(
                    table_hbm.at[idx_v.at[chunk * KSUB + j]],
                    buf.at[pl.ds(j * IDX_ROW, IDX_ROW)],
                    sem,
                ).wait()

        def fire_store(chunk, buf, sem):
            pltpu.async_copy(
                buf, out_hbm.at[pl.ds(out_base + chunk * CHUNK, CHUNK)], sem
            )

        def wait_store(chunk, buf, sem):
            pltpu.make_async_copy(
                buf, out_hbm.at[pl.ds(out_base + chunk * CHUNK, CHUNK)], sem
            ).wait()

        fire_gather(0, rows0, gs0)

        t_steps = g_steps // 2

        def body(t, carry):
            a = 2 * t
            b = a + 1

            @pl.when(t > 0)
            def _():
                wait_store(a - 1, rows1, ss1)

            fire_gather(b, rows1, gs1)
            wait_gather(a, rows0, gs0)
            fire_store(a, rows0, ss0)

            @pl.when(t + 1 < t_steps)
            def _():
                wait_store(a, rows0, ss0)
                fire_gather(a + 2, rows0, gs0)

            wait_gather(b, rows1, gs1)
            fire_store(b, rows1, ss1)
            return carry

        lax.fori_loop(0, t_steps, body, 0)
        wait_store(g_steps - 2, rows0, ss0)
        wait_store(g_steps - 1, rows1, ss1)

    return k(weight, idx_rows)


def kernel(token_ids, weight):
    b, s = token_ids.shape
    b_total = b * s
    n = weight.shape[0]
    n_full = (n // K1_COLS) * K1_COLS
    tail_tab = jnp.pad(
        weight[n_full:], ((0, K1_COLS - (n - n_full)), (0, EMB_D))
    )
    table = _expand_table(weight.T, tail_tab)
    wpad = table.reshape(2 * table.shape[0], EMB_D)
    idx_rows = (token_ids.astype(jnp.int32) * 2).reshape(b_total // IDX_ROW, IDX_ROW)
    out = _gather_sc(wpad, idx_rows, b_total)
    return out.reshape(b, s, EMB_D)


# speedup vs baseline: 1.2336x; 1.0369x over previous
import functools

import jax
import jax.numpy as jnp
from jax import lax
from jax.experimental import pallas as pl
from jax.experimental.pallas import tpu as pltpu
from jax.experimental.pallas import tpu_sc as plsc

EMB_D = 64
IDX_ROW = 128
KSUB = 4
CHUNK = KSUB * IDX_ROW


K1_COLS = 16384


def _expand_table(weight_t, tail_tab):
    n = weight_t.shape[1]
    n_full = (n // K1_COLS) * K1_COLS
    steps = n_full // K1_COLS + 1

    def body(in_ref, tail_ref, out_ref):
        r = pl.program_id(0)

        @pl.when(r < steps - 1)
        def _():
            t = in_ref[...].T
            out_ref[...] = jnp.concatenate([t, t], axis=1)

        @pl.when(r == steps - 1)
        def _():
            out_ref[...] = tail_ref[...]

    return pl.pallas_call(
        body,
        grid=(steps,),
        in_specs=[
            pl.BlockSpec(
                (EMB_D, K1_COLS), lambda r: (0, jnp.minimum(r, steps - 2))
            ),
            pl.BlockSpec((K1_COLS, 2 * EMB_D), lambda r: (0, 0)),
        ],
        out_specs=pl.BlockSpec((K1_COLS, 2 * EMB_D), lambda r: (r, 0)),
        out_shape=jax.ShapeDtypeStruct((steps * K1_COLS, 2 * EMB_D), jnp.float32),
    )(weight_t, tail_tab)


def _gather_sc(weight, idx_rows, b_total):
    info = plsc.get_sparse_core_info()
    nc, ns = info.num_cores, info.num_subcores
    nw = nc * ns
    rows_per_w = b_total // nw
    g_steps = rows_per_w // CHUNK
    irows_per_w = rows_per_w // IDX_ROW

    mesh = plsc.VectorSubcoreMesh(core_axis_name="c", subcore_axis_name="s")

    @functools.partial(
        pl.kernel,
        mesh=mesh,
        compiler_params=pltpu.CompilerParams(use_tc_tiling_on_sc=False),
        out_type=jax.ShapeDtypeStruct((b_total, EMB_D), jnp.float32),
        scratch_types=[
            pltpu.VMEM((irows_per_w, IDX_ROW), jnp.int32),
            pltpu.VMEM((CHUNK, EMB_D), jnp.float32),
            pltpu.VMEM((CHUNK, EMB_D), jnp.float32),
            pltpu.SemaphoreType.DMA,
            pltpu.SemaphoreType.DMA,
            pltpu.SemaphoreType.DMA,
            pltpu.SemaphoreType.DMA,
        ],
    )
    def k(table_hbm, idx_hbm, out_hbm, idx_v, rows0, rows1, gs0, gs1, ss0, ss1):
        wid = lax.axis_index("s") * nc + lax.axis_index("c")
        irow_base = wid * irows_per_w
        out_base = wid * rows_per_w

        pltpu.sync_copy(idx_hbm.at[pl.ds(irow_base, irows_per_w)], idx_v)

        def fire_gather(chunk, buf, sem):
            for j in range(KSUB):
                pltpu.async_copy(
                    table_hbm.at[idx_v.at[chunk * KSUB + j]],
                    buf.at[pl.ds(j * IDX_ROW, IDX_ROW)],
                    sem,
                )

        def wait_gather(chunk, buf, sem):
            for j in range(KSUB):
                pltpu.make_async_copy(
                    table_hbm.at[idx_v.at[chunk * KSUB + j]],
                    buf.at[pl.ds(j * IDX_ROW, IDX_ROW)],
                    sem,
                ).wait()

        def fire_store(chunk, buf, sem):
            pltpu.async_copy(
                buf, out_hbm.at[pl.ds(out_base + chunk * CHUNK, CHUNK)], sem
            )

        def wait_store(chunk, buf, sem):
            pltpu.make_async_copy(
                buf, out_hbm.at[pl.ds(out_base + chunk * CHUNK, CHUNK)], sem
            ).wait()

        fire_gather(0, rows0, gs0)

        t_steps = g_steps // 2

        def body(t, carry):
            a = 2 * t
            b = a + 1

            @pl.when(t > 0)
            def _():
                wait_store(a - 1, rows1, ss1)

            fire_gather(b, rows1, gs1)
            wait_gather(a, rows0, gs0)
            fire_store(a, rows0, ss0)

            @pl.when(t + 1 < t_steps)
            def _():
                wait_store(a, rows0, ss0)
                fire_gather(a + 2, rows0, gs0)

            wait_gather(b, rows1, gs1)
            fire_store(b, rows1, ss1)
            return carry

        lax.fori_loop(0, t_steps, body, 0)
        wait_store(g_steps - 2, rows0, ss0)
        wait_store(g_steps - 1, rows1, ss1)

    return k(weight, idx_rows)


def kernel(token_ids, weight):
    b, s = token_ids.shape
    b_total = b * s
    n = weight.shape[0]
    n_full = (n // K1_COLS) * K1_COLS
    tail_tab = jnp.pad(
        weight[n_full:], ((0, K1_COLS - (n - n_full)), (0, EMB_D))
    )
    table = _expand_table(weight.T, tail_tab)
    wpad = table.reshape(2 * table.shape[0], EMB_D)
    idx_rows = (token_ids.astype(jnp.int32) * 2).reshape(b_total // IDX_ROW, IDX_ROW)
    out = _gather_sc(wpad, idx_rows, b_total)
    return out.reshape(b, s, EMB_D)
